# Initial kernel scaffold; baseline (speedup 1.0000x reference)
import jax
import jax.numpy as jnp
from jax.experimental import pallas as pl


def kernel(x, edge_index, W_in, b_in, Wc, bc, Wl, bl, gamma, beta, W1, b1, W2, b2):
    raise NotImplementedError("write your pallas kernel here")


# speedup vs baseline: 11.9885x; 11.9885x over previous
import functools

import jax
import jax.numpy as jnp
from jax import lax
from jax.experimental import pallas as pl
from jax.experimental.pallas import tpu as pltpu
from jax.experimental.pallas import tpu_sc as plsc

_N = 10000
_E = 320000
_D = 128
_H2 = 64
_C = 2
_L = 4
_ALPHA = 0.1
_THETA = 0.5
_REST = 1.0 - _ALPHA - _THETA
_EPS = 1e-5

_NC = 2
_NS = 16
_NW = _NC * _NS
_CHUNK = 128
_EPW = _E // _NW
_CPW = 80
_TPW = _CPW * _CHUNK
_PADE = _TPW - _EPW
_PADROWS = 240
_NA = _N + _PADROWS
_RPT = 640
_RTAIL = _N - 15 * _RPT
_DPT = _NA // _NS

_BN = 1000
_NB = _N // _BN


def _rsqrt_exact(v):
    return lax.rsqrt(v)

@functools.cache
def _sc_mesh():
    return plsc.VectorSubcoreMesh(core_axis_name="c", subcore_axis_name="s",
                                  num_cores=_NC, num_subcores=_NS)


def _deg_body(dstp_hbm, out_hbm, acc, idx, ones, zbuf):
    c = lax.axis_index("c")
    t = lax.axis_index("s")

    def fill_z(j, carry):
        zbuf[pl.ds(j * 16, 16)] = jnp.zeros((16,), jnp.float32)
        return carry

    lax.fori_loop(0, _DPT // 16, fill_z, 0)
    for k in range(_CHUNK // 16):
        ones[pl.ds(k * 16, 16)] = jnp.ones((16,), jnp.float32)
    pltpu.sync_copy(zbuf, acc.at[pl.ds(t * _DPT, _DPT)])
    plsc.subcore_barrier()

    w = c * _NS + t

    def step(j, carry):
        base = w * _TPW + j * _CHUNK
        pltpu.sync_copy(dstp_hbm.at[pl.ds(base, _CHUNK)], idx)
        pltpu.sync_copy(ones, acc.at[idx], add=True)
        return carry

    lax.fori_loop(0, _CPW, step, 0)
    plsc.subcore_barrier()
    pltpu.sync_copy(acc.at[pl.ds(t * _DPT, _DPT)],
                    out_hbm.at[c, pl.ds(t * _DPT, _DPT)])


@functools.cache
def _deg_kernel():
    return pl.kernel(
        _deg_body,
        out_type=jax.ShapeDtypeStruct((_NC, _NA), jnp.float32),
        mesh=_sc_mesh(),
        scratch_types=[
            pltpu.VMEM_SHARED((_NA,), jnp.float32),
            pltpu.VMEM((_CHUNK,), jnp.int32),
            pltpu.VMEM((_CHUNK,), jnp.float32),
            pltpu.VMEM((_DPT,), jnp.float32),
        ],
    )


def _deg_call(dstp):
    return _deg_kernel()(dstp)


def _spmm_body(mp_hbm, srcp_hbm, dstp_hbm, out_hbm, acc, idxs, idxd, rows, sem):
    c = lax.axis_index("c")
    t = lax.axis_index("s")

    @pl.when(t < _NS - 1)
    def _():
        pltpu.sync_copy(mp_hbm.at[pl.ds(t * _RPT, _RPT)],
                        acc.at[pl.ds(t * _RPT, _RPT)])

    @pl.when(t == _NS - 1)
    def _():
        pltpu.sync_copy(mp_hbm.at[pl.ds(15 * _RPT, _RTAIL)],
                        acc.at[pl.ds(15 * _RPT, _RTAIL)])

    plsc.subcore_barrier()

    w = c * _NS + t

    def step(j, carry):
        base = w * _TPW + j * _CHUNK
        pltpu.sync_copy(srcp_hbm.at[pl.ds(base, _CHUNK)], idxs)
        pltpu.sync_copy(dstp_hbm.at[pl.ds(base, _CHUNK)], idxd)
        pltpu.async_copy(mp_hbm.at[idxs], rows, sem).wait()
        pltpu.sync_copy(rows, acc.at[idxd], add=True)
        return carry

    lax.fori_loop(0, _CPW, step, 0)
    plsc.subcore_barrier()

    @pl.when(t < _NS - 1)
    def _():
        pltpu.sync_copy(acc.at[pl.ds(t * _RPT, _RPT)],
                        out_hbm.at[c, pl.ds(t * _RPT, _RPT)])

    @pl.when(t == _NS - 1)
    def _():
        pltpu.sync_copy(acc.at[pl.ds(15 * _RPT, _RTAIL)],
                        out_hbm.at[c, pl.ds(15 * _RPT, _RTAIL)])


@functools.cache
def _spmm_kernel():
    return pl.kernel(
        _spmm_body,
        out_type=jax.ShapeDtypeStruct((_NC, _N, _D), jnp.float32),
        mesh=_sc_mesh(),
        scratch_types=[
            pltpu.VMEM_SHARED((_NA, _D), jnp.float32),
            pltpu.VMEM((_CHUNK,), jnp.int32),
            pltpu.VMEM((_CHUNK,), jnp.int32),
            pltpu.VMEM((_CHUNK, _D), jnp.float32),
            pltpu.SemaphoreType.DMA,
        ],
    )


def _spmm_call(mp, srcp, dstp):
    return _spmm_kernel()(mp, srcp, dstp)


def _tc_pre_body(x_ref, win_ref, bin_ref, wc_ref, d0_ref, d1_ref,
                 h0_ref, dinv_ref, mp_ref):
    h = jnp.dot(x_ref[...], win_ref[...], preferred_element_type=jnp.float32)
    h = jnp.maximum(h + bin_ref[...], 0.0)
    h0_ref[...] = h
    dinv = _rsqrt_exact(d0_ref[...] + d1_ref[...] + 1.0)
    dinv_ref[...] = dinv
    m = jnp.dot(h, wc_ref[...], preferred_element_type=jnp.float32)
    mp_ref[...] = m * dinv


def _tc_pre(x, w_in, b_in, wc0, d0, d1):
    return pl.pallas_call(
        _tc_pre_body,
        grid=(_NB,),
        in_specs=[
            pl.BlockSpec((_BN, _D), lambda i: (i, 0)),
            pl.BlockSpec((_D, _D), lambda i: (0, 0)),
            pl.BlockSpec((1, _D), lambda i: (0, 0)),
            pl.BlockSpec((_D, _D), lambda i: (0, 0)),
            pl.BlockSpec((_BN, 1), lambda i: (i, 0)),
            pl.BlockSpec((_BN, 1), lambda i: (i, 0)),
        ],
        out_specs=[
            pl.BlockSpec((_BN, _D), lambda i: (i, 0)),
            pl.BlockSpec((_BN, 1), lambda i: (i, 0)),
            pl.BlockSpec((_BN, _D), lambda i: (i, 0)),
        ],
        out_shape=[
            jax.ShapeDtypeStruct((_N, _D), jnp.float32),
            jax.ShapeDtypeStruct((_N, 1), jnp.float32),
            jax.ShapeDtypeStruct((_N, _D), jnp.float32),
        ],
    )(x, w_in, b_in, wc0, d0, d1)


def _tc_mid_body(s_ref, mp_ref, dinv_ref, bc_ref, wl_ref, bl_ref,
                 p_ref, stats_ref, stats_acc):
    i = pl.program_id(0)

    @pl.when(i == 0)
    def _():
        stats_acc[...] = jnp.zeros_like(stats_acc)

    s_sum = s_ref[0] + s_ref[1] - mp_ref[...]
    agg = s_sum * dinv_ref[...] + bc_ref[...]
    p = jnp.dot(agg, wl_ref[...], preferred_element_type=jnp.float32)
    p = p + bl_ref[...]
    p_ref[...] = p
    stats_acc[0:1] = stats_acc[0:1] + jnp.sum(p, axis=0, keepdims=True)
    stats_acc[1:2] = stats_acc[1:2] + jnp.sum(p * p, axis=0, keepdims=True)

    @pl.when(i == _NB - 1)
    def _():
        stats_ref[...] = stats_acc[...]


def _tc_mid(s, mp, dinv, bc, wl, bl):
    return pl.pallas_call(
        _tc_mid_body,
        grid=(_NB,),
        in_specs=[
            pl.BlockSpec((_NC, _BN, _D), lambda i: (0, i, 0)),
            pl.BlockSpec((_BN, _D), lambda i: (i, 0)),
            pl.BlockSpec((_BN, 1), lambda i: (i, 0)),
            pl.BlockSpec((1, _D), lambda i: (0, 0)),
            pl.BlockSpec((_D, _D), lambda i: (0, 0)),
            pl.BlockSpec((1, _D), lambda i: (0, 0)),
        ],
        out_specs=[
            pl.BlockSpec((_BN, _D), lambda i: (i, 0)),
            pl.BlockSpec((2, _D), lambda i: (0, 0)),
        ],
        out_shape=[
            jax.ShapeDtypeStruct((_N, _D), jnp.float32),
            jax.ShapeDtypeStruct((2, _D), jnp.float32),
        ],
        scratch_shapes=[pltpu.VMEM((2, _D), jnp.float32)],
    )(s, mp, dinv, bc, wl, bl)


def _bn_mix(p_ref, stats_ref, g_ref, b_ref, h0_ref, hp_ref):
    mean = stats_ref[0:1] / _N
    var = stats_ref[1:2] / _N - mean * mean
    k = g_ref[...] * _rsqrt_exact(var + _EPS)
    shift = b_ref[...] - mean * k
    hn = p_ref[...] * k + shift
    return jnp.maximum(_REST * hn + _ALPHA * h0_ref[...] + _THETA * hp_ref[...],
                       0.0)


def _tc_post_body(p_ref, stats_ref, g_ref, b_ref, h0_ref, hp_ref, wc_ref,
                  dinv_ref, h_ref, mp_ref):
    h = _bn_mix(p_ref, stats_ref, g_ref, b_ref, h0_ref, hp_ref)
    h_ref[...] = h
    m = jnp.dot(h, wc_ref[...], preferred_element_type=jnp.float32)
    mp_ref[...] = m * dinv_ref[...]


def _tc_post(p, stats, g, b, h0, hp, wc_next, dinv):
    return pl.pallas_call(
        _tc_post_body,
        grid=(_NB,),
        in_specs=[
            pl.BlockSpec((_BN, _D), lambda i: (i, 0)),
            pl.BlockSpec((2, _D), lambda i: (0, 0)),
            pl.BlockSpec((1, _D), lambda i: (0, 0)),
            pl.BlockSpec((1, _D), lambda i: (0, 0)),
            pl.BlockSpec((_BN, _D), lambda i: (i, 0)),
            pl.BlockSpec((_BN, _D), lambda i: (i, 0)),
            pl.BlockSpec((_D, _D), lambda i: (0, 0)),
            pl.BlockSpec((_BN, 1), lambda i: (i, 0)),
        ],
        out_specs=[
            pl.BlockSpec((_BN, _D), lambda i: (i, 0)),
            pl.BlockSpec((_BN, _D), lambda i: (i, 0)),
        ],
        out_shape=[
            jax.ShapeDtypeStruct((_N, _D), jnp.float32),
            jax.ShapeDtypeStruct((_N, _D), jnp.float32),
        ],
    )(p, stats, g, b, h0, hp, wc_next, dinv)


def _tc_last_body(p_ref, stats_ref, g_ref, b_ref, h0_ref, hp_ref,
                  w1_ref, b1_ref, w2_ref, b2_ref, out_ref):
    h = _bn_mix(p_ref, stats_ref, g_ref, b_ref, h0_ref, hp_ref)
    z = jnp.dot(h, w1_ref[...], preferred_element_type=jnp.float32)
    z = jnp.maximum(z + b1_ref[...], 0.0)
    o = jnp.dot(z, w2_ref[...], preferred_element_type=jnp.float32)
    out_ref[...] = o + b2_ref[...]


def _tc_last(p, stats, g, b, h0, hp, w1, b1, w2, b2):
    return pl.pallas_call(
        _tc_last_body,
        grid=(_NB,),
        in_specs=[
            pl.BlockSpec((_BN, _D), lambda i: (i, 0)),
            pl.BlockSpec((2, _D), lambda i: (0, 0)),
            pl.BlockSpec((1, _D), lambda i: (0, 0)),
            pl.BlockSpec((1, _D), lambda i: (0, 0)),
            pl.BlockSpec((_BN, _D), lambda i: (i, 0)),
            pl.BlockSpec((_BN, _D), lambda i: (i, 0)),
            pl.BlockSpec((_D, _H2), lambda i: (0, 0)),
            pl.BlockSpec((1, _H2), lambda i: (0, 0)),
            pl.BlockSpec((_H2, _C), lambda i: (0, 0)),
            pl.BlockSpec((1, _C), lambda i: (0, 0)),
        ],
        out_specs=[pl.BlockSpec((_BN, _C), lambda i: (i, 0))],
        out_shape=[jax.ShapeDtypeStruct((_N, _C), jnp.float32)],
    )(p, stats, g, b, h0, hp, w1, b1, w2, b2)[0]


def kernel(x, edge_index, W_in, b_in, Wc, bc, Wl, bl, gamma, beta, W1, b1,
           W2, b2):
    src = edge_index[0]
    dst = edge_index[1]
    padk = jnp.arange(_NW * _PADE, dtype=jnp.int32).reshape(_NW, _PADE)
    srcp = jnp.concatenate(
        [src.reshape(_NW, _EPW), padk % _N], axis=1).reshape(-1)
    dstp = jnp.concatenate(
        [dst.reshape(_NW, _EPW), _N + padk % _PADROWS], axis=1).reshape(-1)

    degp = _deg_call(dstp)
    d0 = degp[0, :_N].reshape(_N, 1)
    d1 = degp[1, :_N].reshape(_N, 1)

    b_in2 = b_in.reshape(1, _D)
    h0, dinv, mp = _tc_pre(x, W_in, b_in2, Wc[0], d0, d1)

    hp = h0
    for i in range(_L):
        s = _spmm_call(mp, srcp, dstp)
        p, stats = _tc_mid(s, mp, dinv, bc[i].reshape(1, _D), Wl[i],
                           bl[i].reshape(1, _D))
        g2 = gamma[i].reshape(1, _D)
        be2 = beta[i].reshape(1, _D)
        if i < _L - 1:
            hp, mp = _tc_post(p, stats, g2, be2, h0, hp, Wc[i + 1], dinv)
        else:
            out = _tc_last(p, stats, g2, be2, h0, hp, W1,
                           b1.reshape(1, _H2), W2, b2.reshape(1, _C))
    return out
